# Initial kernel scaffold; baseline (speedup 1.0000x reference)
import jax
import jax.numpy as jnp
from jax.experimental import pallas as pl


def kernel(x, conv1_w, bn1_scale, bn1_shift, l0b0_conv1_w, l0b0_bn1_scale, l0b0_bn1_shift, l0b0_conv2_w, l0b0_bn2_scale, l0b0_bn2_shift, l0b1_conv1_w, l0b1_bn1_scale, l0b1_bn1_shift, l0b1_conv2_w, l0b1_bn2_scale, l0b1_bn2_shift, l0b2_conv1_w, l0b2_bn1_scale, l0b2_bn1_shift, l0b2_conv2_w, l0b2_bn2_scale, l0b2_bn2_shift, l1b0_conv1_w, l1b0_bn1_scale, l1b0_bn1_shift, l1b0_conv2_w, l1b0_bn2_scale, l1b0_bn2_shift, l1b0_down_w, l1b0_down_scale, l1b0_down_shift, l1b1_conv1_w, l1b1_bn1_scale, l1b1_bn1_shift, l1b1_conv2_w, l1b1_bn2_scale, l1b1_bn2_shift, l1b2_conv1_w, l1b2_bn1_scale, l1b2_bn1_shift, l1b2_conv2_w, l1b2_bn2_scale, l1b2_bn2_shift, l1b3_conv1_w, l1b3_bn1_scale, l1b3_bn1_shift, l1b3_conv2_w, l1b3_bn2_scale, l1b3_bn2_shift, l2b0_conv1_w, l2b0_bn1_scale, l2b0_bn1_shift, l2b0_conv2_w, l2b0_bn2_scale, l2b0_bn2_shift, l2b0_down_w, l2b0_down_scale, l2b0_down_shift, l2b1_conv1_w, l2b1_bn1_scale, l2b1_bn1_shift, l2b1_conv2_w, l2b1_bn2_scale, l2b1_bn2_shift, l2b2_conv1_w, l2b2_bn1_scale, l2b2_bn1_shift, l2b2_conv2_w, l2b2_bn2_scale, l2b2_bn2_shift, l2b3_conv1_w, l2b3_bn1_scale, l2b3_bn1_shift, l2b3_conv2_w, l2b3_bn2_scale, l2b3_bn2_shift, l2b4_conv1_w, l2b4_bn1_scale, l2b4_bn1_shift, l2b4_conv2_w, l2b4_bn2_scale, l2b4_bn2_shift, l2b5_conv1_w, l2b5_bn1_scale, l2b5_bn1_shift, l2b5_conv2_w, l2b5_bn2_scale, l2b5_bn2_shift, l3b0_conv1_w, l3b0_bn1_scale, l3b0_bn1_shift, l3b0_conv2_w, l3b0_bn2_scale, l3b0_bn2_shift, l3b0_down_w, l3b0_down_scale, l3b0_down_shift, l3b1_conv1_w, l3b1_bn1_scale, l3b1_bn1_shift, l3b1_conv2_w, l3b1_bn2_scale, l3b1_bn2_shift, l3b2_conv1_w, l3b2_bn1_scale, l3b2_bn1_shift, l3b2_conv2_w, l3b2_bn2_scale, l3b2_bn2_shift, fc0_w, fc0_b, fc1_w, fc1_b, fc2_w, fc2_b, fc3_w, fc3_b):
    raise NotImplementedError("write your pallas kernel here")


# speedup vs baseline: 2.0055x; 2.0055x over previous
import functools

import jax
import jax.numpy as jnp
from jax.experimental import pallas as pl
from jax.experimental.pallas import tpu as pltpu

_VMEM_LIMIT = 64 * 1024 * 1024


def _rnd_up(v, m):
    return ((v + m - 1) // m) * m


def _conv3s1_body(x_ref, w_ref, s_ref, c_ref, *rest, h, w_real, w_pad,
                  relu, has_res):
    if has_res:
        r_ref, o_ref, acc_ref = rest
    else:
        o_ref, acc_ref = rest
    bt = x_ref.shape[0]
    cin = x_ref.shape[-1]
    co = o_ref.shape[-1]
    m = bt * h * w_pad
    for di in range(3):
        for dj in range(3):
            a = x_ref[:, di:di + h, dj:dj + w_pad, :].reshape(m, cin)
            wt = w_ref[(di * 3 + dj) * cin:(di * 3 + dj + 1) * cin, :]
            d = jnp.dot(a, wt, preferred_element_type=jnp.float32)
            if di == 0 and dj == 0:
                acc_ref[...] = d
            else:
                acc_ref[...] += d
    out = acc_ref[...] * s_ref[...] + c_ref[...]
    if has_res:
        out = out + r_ref[...].reshape(m, co).astype(jnp.float32)
    if relu:
        out = jnp.maximum(out, 0.0)
    if w_real != w_pad:
        col = jax.lax.broadcasted_iota(jnp.int32, (m, 1), 0) % w_pad
        out = jnp.where(col < w_real, out, 0.0)
    o_ref[...] = out.astype(o_ref.dtype).reshape(bt, h, w_pad, co)


def _conv3x3_s1(x, w2d, scale, shift, w_real, residual=None, relu=True, bt=1):
    n, h, w_pad, cin = x.shape
    kp, co = w2d.shape
    xp = jnp.pad(x, ((0, 0), (1, 1), (1, 1), (0, 0)))
    has_res = residual is not None
    in_specs = [
        pl.BlockSpec((bt, h + 2, w_pad + 2, cin), lambda i: (i, 0, 0, 0)),
        pl.BlockSpec((kp, co), lambda i: (0, 0)),
        pl.BlockSpec((1, co), lambda i: (0, 0)),
        pl.BlockSpec((1, co), lambda i: (0, 0)),
    ]
    args = [xp, w2d, scale, shift]
    if has_res:
        in_specs.append(pl.BlockSpec((bt, h, w_pad, co),
                                     lambda i: (i, 0, 0, 0)))
        args.append(residual)
    m = bt * h * w_pad
    flops = 2 * n * h * w_pad * kp * co
    bytes_accessed = (n * (h + 2) * (w_pad + 2) * cin * 2 + kp * co * 2
                      + n * h * w_pad * co * (4 if has_res else 2))
    return pl.pallas_call(
        functools.partial(_conv3s1_body, h=h, w_real=w_real, w_pad=w_pad,
                          relu=relu, has_res=has_res),
        out_shape=jax.ShapeDtypeStruct((n, h, w_pad, co), jnp.bfloat16),
        grid=(n // bt,),
        in_specs=in_specs,
        out_specs=pl.BlockSpec((bt, h, w_pad, co), lambda i: (i, 0, 0, 0)),
        scratch_shapes=[pltpu.VMEM((m, co), jnp.float32)],
        compiler_params=pltpu.CompilerParams(
            dimension_semantics=("parallel",),
            vmem_limit_bytes=_VMEM_LIMIT),
        cost_estimate=pl.CostEstimate(flops=flops, transcendentals=0,
                                      bytes_accessed=bytes_accessed),
    )(*args)


def _mm_body(a_ref, b_ref, s_ref, c_ref, o_ref, *, relu):
    out = jnp.dot(a_ref[...], b_ref[...], preferred_element_type=jnp.float32)
    out = out * s_ref[...] + c_ref[...]
    if relu:
        out = jnp.maximum(out, 0.0)
    o_ref[...] = out.astype(o_ref.dtype)


def _matmul_bn(a, b, scale, shift, relu=True, tm=512):
    m, k = a.shape
    kb, nb = b.shape
    mp = _rnd_up(m, tm)
    if mp != m:
        a = jnp.pad(a, ((0, mp - m), (0, 0)))
    flops = 2 * mp * kb * nb
    bytes_accessed = mp * k * 2 + kb * nb * 2 + mp * nb * 2
    out = pl.pallas_call(
        functools.partial(_mm_body, relu=relu),
        out_shape=jax.ShapeDtypeStruct((mp, nb), jnp.bfloat16),
        grid=(mp // tm,),
        in_specs=[
            pl.BlockSpec((tm, k), lambda i: (i, 0)),
            pl.BlockSpec((kb, nb), lambda i: (0, 0)),
            pl.BlockSpec((1, nb), lambda i: (0, 0)),
            pl.BlockSpec((1, nb), lambda i: (0, 0)),
        ],
        out_specs=pl.BlockSpec((tm, nb), lambda i: (i, 0)),
        compiler_params=pltpu.CompilerParams(
            dimension_semantics=("parallel",),
            vmem_limit_bytes=_VMEM_LIMIT),
        cost_estimate=pl.CostEstimate(flops=flops, transcendentals=0,
                                      bytes_accessed=bytes_accessed),
    )(a, b, scale, shift)
    return out[:m] if mp != m else out


def _maxpool_body(e_ref, o_ref, out_ref, *, ho, wo):
    e = e_ref[0]
    o = o_ref[0]
    wm = jnp.maximum(jnp.maximum(e[:, 0:wo, :], o[:, 0:wo, :]),
                     e[:, 1:wo + 1, :])
    pa = wm[0:2 * ho].reshape(ho, 2, wo, wm.shape[-1])
    pb = wm[2:2 * ho + 2].reshape(ho, 2, wo, wm.shape[-1])
    out_ref[0] = jnp.maximum(jnp.maximum(pa[:, 0], pa[:, 1]), pb[:, 0])


def _maxpool_3x3_s2(x):
    n, h, w, c = x.shape
    ho, wo = h // 2, w // 2
    xp = jnp.pad(x, ((0, 0), (1, 1), (1, 1), (0, 0)))
    ev = xp[:, :, 0::2, :]
    od = xp[:, :, 1::2, :]
    hp, wp = h + 2, w // 2 + 1
    spec = pl.BlockSpec((1, hp, wp, c), lambda i: (i, 0, 0, 0))
    return pl.pallas_call(
        functools.partial(_maxpool_body, ho=ho, wo=wo),
        out_shape=jax.ShapeDtypeStruct((n, ho, wo, c), x.dtype),
        grid=(n,),
        in_specs=[spec, spec],
        out_specs=pl.BlockSpec((1, ho, wo, c), lambda i: (i, 0, 0, 0)),
        compiler_params=pltpu.CompilerParams(
            dimension_semantics=("parallel",),
            vmem_limit_bytes=_VMEM_LIMIT),
    )(ev, od)


def _tail_body(x_ref, w1_ref, b1_ref, w2_ref, b2_ref, w3_ref, b3_ref,
               w4_ref, b4_ref, out_ref, feat_ref, *, inv_hw):
    feats = jnp.sum(x_ref[...].astype(jnp.float32), axis=1) * inv_hw
    feat_ref[...] = feats
    h = feats
    for w_ref, b_ref in ((w1_ref, b1_ref), (w2_ref, b2_ref),
                         (w3_ref, b3_ref), (w4_ref, b4_ref)):
        h = jnp.dot(h.astype(w_ref.dtype), w_ref[...],
                    preferred_element_type=jnp.float32) + b_ref[...]
    ss = jnp.sum(h * h, axis=-1, keepdims=True)
    out_ref[...] = h * jax.lax.rsqrt(ss + 1e-12)


def _tail(x3, fcs, hw_real):
    n = x3.shape[0]
    c = x3.shape[-1]
    (w1, b1), (w2, b2), (w3, b3), (w4, b4) = fcs
    out, feats = pl.pallas_call(
        functools.partial(_tail_body, inv_hw=1.0 / float(hw_real)),
        out_shape=(jax.ShapeDtypeStruct((n, w4.shape[1]), jnp.float32),
                   jax.ShapeDtypeStruct((n, c), jnp.float32)),
        compiler_params=pltpu.CompilerParams(vmem_limit_bytes=_VMEM_LIMIT),
    )(x3, w1, b1, w2, b2, w3, b3, w4, b4)
    return out, feats


def _im2col3_s2(x, w_real):
    n, h, _, c = x.shape
    x = x[:, :, :w_real, :]
    ho, wo = h // 2, w_real // 2
    xp = jnp.pad(x, ((0, 0), (1, 1), (1, 1), (0, 0)))
    views = []
    for i in range(3):
        for j in range(3):
            views.append(xp[:, i:i + 2 * ho:2, j:j + 2 * wo:2, :])
    cols = jnp.stack(views, axis=3).reshape(n * ho * wo, 9 * c)
    return cols, ho, wo


def _to_padded_map(flat, n, ho, wo, co):
    x = flat.reshape(n, ho, wo, co)
    wp = _rnd_up(wo, 8)
    if wp != wo:
        x = jnp.pad(x, ((0, 0), (0, 0), (0, wp - wo), (0, 0)))
    return x


def kernel(x, conv1_w, bn1_scale, bn1_shift, l0b0_conv1_w, l0b0_bn1_scale, l0b0_bn1_shift, l0b0_conv2_w, l0b0_bn2_scale, l0b0_bn2_shift, l0b1_conv1_w, l0b1_bn1_scale, l0b1_bn1_shift, l0b1_conv2_w, l0b1_bn2_scale, l0b1_bn2_shift, l0b2_conv1_w, l0b2_bn1_scale, l0b2_bn1_shift, l0b2_conv2_w, l0b2_bn2_scale, l0b2_bn2_shift, l1b0_conv1_w, l1b0_bn1_scale, l1b0_bn1_shift, l1b0_conv2_w, l1b0_bn2_scale, l1b0_bn2_shift, l1b0_down_w, l1b0_down_scale, l1b0_down_shift, l1b1_conv1_w, l1b1_bn1_scale, l1b1_bn1_shift, l1b1_conv2_w, l1b1_bn2_scale, l1b1_bn2_shift, l1b2_conv1_w, l1b2_bn1_scale, l1b2_bn1_shift, l1b2_conv2_w, l1b2_bn2_scale, l1b2_bn2_shift, l1b3_conv1_w, l1b3_bn1_scale, l1b3_bn1_shift, l1b3_conv2_w, l1b3_bn2_scale, l1b3_bn2_shift, l2b0_conv1_w, l2b0_bn1_scale, l2b0_bn1_shift, l2b0_conv2_w, l2b0_bn2_scale, l2b0_bn2_shift, l2b0_down_w, l2b0_down_scale, l2b0_down_shift, l2b1_conv1_w, l2b1_bn1_scale, l2b1_bn1_shift, l2b1_conv2_w, l2b1_bn2_scale, l2b1_bn2_shift, l2b2_conv1_w, l2b2_bn1_scale, l2b2_bn1_shift, l2b2_conv2_w, l2b2_bn2_scale, l2b2_bn2_shift, l2b3_conv1_w, l2b3_bn1_scale, l2b3_bn1_shift, l2b3_conv2_w, l2b3_bn2_scale, l2b3_bn2_shift, l2b4_conv1_w, l2b4_bn1_scale, l2b4_bn1_shift, l2b4_conv2_w, l2b4_bn2_scale, l2b4_bn2_shift, l2b5_conv1_w, l2b5_bn1_scale, l2b5_bn1_shift, l2b5_conv2_w, l2b5_bn2_scale, l2b5_bn2_shift, l3b0_conv1_w, l3b0_bn1_scale, l3b0_bn1_shift, l3b0_conv2_w, l3b0_bn2_scale, l3b0_bn2_shift, l3b0_down_w, l3b0_down_scale, l3b0_down_shift, l3b1_conv1_w, l3b1_bn1_scale, l3b1_bn1_shift, l3b1_conv2_w, l3b1_bn2_scale, l3b1_bn2_shift, l3b2_conv1_w, l3b2_bn1_scale, l3b2_bn1_shift, l3b2_conv2_w, l3b2_bn2_scale, l3b2_bn2_shift, fc0_w, fc0_b, fc1_w, fc1_b, fc2_w, fc2_b, fc3_w, fc3_b):
    v = dict(locals())
    n = x.shape[0]

    xh = jnp.transpose(x, (0, 2, 3, 1)).astype(jnp.bfloat16)
    hi = xh.shape[1]
    ho1 = hi // 2
    xp = jnp.pad(xh, ((0, 0), (3, 3), (3, 3), (0, 0)))
    views = [xp[:, i:i + 2 * ho1:2, j:j + 2 * ho1:2, :]
             for i in range(7) for j in range(7)]
    cols = jnp.stack(views, axis=3).reshape(n * ho1 * ho1, 49 * xh.shape[-1])
    kr = cols.shape[1]
    h1 = _matmul_bn(cols, conv1_w[:kr], bn1_scale, bn1_shift, relu=True,
                    tm=1024)
    h1 = h1.reshape(n, ho1, ho1, conv1_w.shape[1])

    cur = _maxpool_3x3_s2(h1)
    w_real = cur.shape[2]

    layer_cfg = ((3, 1), (4, 4), (6, 8), (3, 16))
    for li, (nb, bt) in enumerate(layer_cfg):
        for bi in range(nb):
            pfx = "l%db%d_" % (li, bi)
            w1, s1, c1 = v[pfx + "conv1_w"], v[pfx + "bn1_scale"], v[pfx + "bn1_shift"]
            w2, s2, c2 = v[pfx + "conv2_w"], v[pfx + "bn2_scale"], v[pfx + "bn2_shift"]
            if bi == 0 and li > 0:
                cin = cur.shape[-1]
                cols, ho, wo = _im2col3_s2(cur, w_real)
                co = w1.shape[1]
                b1_out = _matmul_bn(cols, w1, s1, c1, relu=True)
                b1_out = _to_padded_map(b1_out, n, ho, wo, co)
                xs = cur[:, ::2, :w_real:2, :].reshape(n * ho * wo, cin)
                idn = _matmul_bn(xs, v[pfx + "down_w"], v[pfx + "down_scale"],
                                 v[pfx + "down_shift"], relu=False)
                idn = _to_padded_map(idn, n, ho, wo, co)
                w_real = wo
            else:
                b1_out = _conv3x3_s1(cur, w1, s1, c1, w_real, relu=True, bt=bt)
                idn = cur
            cur = _conv3x3_s1(b1_out, w2, s2, c2, w_real, residual=idn,
                              relu=True, bt=bt)

    x3 = cur.reshape(n, cur.shape[1] * cur.shape[2], cur.shape[3])
    fcs = [(v["fc%d_w" % i], v["fc%d_b" % i]) for i in range(4)]
    out, feats = _tail(x3, fcs, hw_real=cur.shape[1] * w_real)
    return out[:, :4], feats


# speedup vs baseline: 2.0634x; 1.0289x over previous
import functools

import jax
import jax.numpy as jnp
from jax.experimental import pallas as pl
from jax.experimental.pallas import tpu as pltpu

_VMEM_LIMIT = 64 * 1024 * 1024


def _rnd_up(v, m):
    return ((v + m - 1) // m) * m


def _conv3s1_body(x_ref, w_ref, s_ref, c_ref, *rest, h, w_real, w_pad,
                  relu, has_res):
    if has_res:
        r_ref, o_ref, sh_ref, acc_ref = rest
    else:
        o_ref, sh_ref, acc_ref = rest
    bt = x_ref.shape[0]
    cin = x_ref.shape[-1]
    co = o_ref.shape[-1]
    m = bt * h * w_pad
    for dj in range(3):
        sh_ref[dj] = x_ref[:, :, dj:dj + w_pad, :].reshape(
            bt, (h + 2) * w_pad, cin)
    for di in range(3):
        for dj in range(3):
            a = sh_ref[dj, :, di * w_pad:(di + h) * w_pad, :].reshape(m, cin)
            wt = w_ref[(di * 3 + dj) * cin:(di * 3 + dj + 1) * cin, :]
            d = jnp.dot(a, wt, preferred_element_type=jnp.float32)
            if di == 0 and dj == 0:
                acc_ref[...] = d
            else:
                acc_ref[...] += d
    out = acc_ref[...] * s_ref[...] + c_ref[...]
    if has_res:
        out = out + r_ref[...].reshape(m, co).astype(jnp.float32)
    if relu:
        out = jnp.maximum(out, 0.0)
    if w_real != w_pad:
        col = jax.lax.broadcasted_iota(jnp.int32, (m, 1), 0) % w_pad
        out = jnp.where(col < w_real, out, 0.0)
    o_ref[...] = out.astype(o_ref.dtype).reshape(bt, h, w_pad, co)


def _conv3x3_s1(x, w2d, scale, shift, w_real, residual=None, relu=True, bt=1):
    n, h, w_pad, cin = x.shape
    kp, co = w2d.shape
    xp = jnp.pad(x, ((0, 0), (1, 1), (1, 1), (0, 0)))
    has_res = residual is not None
    in_specs = [
        pl.BlockSpec((bt, h + 2, w_pad + 2, cin), lambda i: (i, 0, 0, 0)),
        pl.BlockSpec((kp, co), lambda i: (0, 0)),
        pl.BlockSpec((1, co), lambda i: (0, 0)),
        pl.BlockSpec((1, co), lambda i: (0, 0)),
    ]
    args = [xp, w2d, scale, shift]
    if has_res:
        in_specs.append(pl.BlockSpec((bt, h, w_pad, co),
                                     lambda i: (i, 0, 0, 0)))
        args.append(residual)
    m = bt * h * w_pad
    flops = 2 * n * h * w_pad * kp * co
    bytes_accessed = (n * (h + 2) * (w_pad + 2) * cin * 2 + kp * co * 2
                      + n * h * w_pad * co * (4 if has_res else 2))
    return pl.pallas_call(
        functools.partial(_conv3s1_body, h=h, w_real=w_real, w_pad=w_pad,
                          relu=relu, has_res=has_res),
        out_shape=jax.ShapeDtypeStruct((n, h, w_pad, co), jnp.bfloat16),
        grid=(n // bt,),
        in_specs=in_specs,
        out_specs=pl.BlockSpec((bt, h, w_pad, co), lambda i: (i, 0, 0, 0)),
        scratch_shapes=[pltpu.VMEM((3, bt, (h + 2) * w_pad, cin),
                                   jnp.bfloat16),
                        pltpu.VMEM((m, co), jnp.float32)],
        compiler_params=pltpu.CompilerParams(
            dimension_semantics=("parallel",),
            vmem_limit_bytes=_VMEM_LIMIT),
        cost_estimate=pl.CostEstimate(flops=flops, transcendentals=0,
                                      bytes_accessed=bytes_accessed),
    )(*args)


def _mm_body(a_ref, b_ref, s_ref, c_ref, o_ref, *, relu):
    out = jnp.dot(a_ref[...], b_ref[...], preferred_element_type=jnp.float32)
    out = out * s_ref[...] + c_ref[...]
    if relu:
        out = jnp.maximum(out, 0.0)
    o_ref[...] = out.astype(o_ref.dtype)


def _matmul_bn(a, b, scale, shift, relu=True, tm=512):
    m, k = a.shape
    kb, nb = b.shape
    mp = _rnd_up(m, tm)
    if mp != m:
        a = jnp.pad(a, ((0, mp - m), (0, 0)))
    flops = 2 * mp * kb * nb
    bytes_accessed = mp * k * 2 + kb * nb * 2 + mp * nb * 2
    out = pl.pallas_call(
        functools.partial(_mm_body, relu=relu),
        out_shape=jax.ShapeDtypeStruct((mp, nb), jnp.bfloat16),
        grid=(mp // tm,),
        in_specs=[
            pl.BlockSpec((tm, k), lambda i: (i, 0)),
            pl.BlockSpec((kb, nb), lambda i: (0, 0)),
            pl.BlockSpec((1, nb), lambda i: (0, 0)),
            pl.BlockSpec((1, nb), lambda i: (0, 0)),
        ],
        out_specs=pl.BlockSpec((tm, nb), lambda i: (i, 0)),
        compiler_params=pltpu.CompilerParams(
            dimension_semantics=("parallel",),
            vmem_limit_bytes=_VMEM_LIMIT),
        cost_estimate=pl.CostEstimate(flops=flops, transcendentals=0,
                                      bytes_accessed=bytes_accessed),
    )(a, b, scale, shift)
    return out[:m] if mp != m else out


def _conv1_body(at_ref, b_ref, s_ref, c_ref, o_ref, *, ho, wo):
    a = at_ref[0]
    out = jax.lax.dot_general(a, b_ref[...], (((0,), (0,)), ((), ())),
                              preferred_element_type=jnp.float32)
    out = jnp.maximum(out * s_ref[...] + c_ref[...], 0.0)
    o_ref[...] = out.astype(o_ref.dtype).reshape(1, ho, wo, out.shape[-1])


def _conv1_7x7_s2(x, w2d, scale, shift):
    n = x.shape[0]
    hi = x.shape[2]
    ho = hi // 2
    kr = 7 * 7 * x.shape[1]
    xp = jnp.pad(x, ((0, 0), (0, 0), (3, 3), (3, 3))).astype(jnp.bfloat16)
    slabs = [xp[:, c, i:i + 2 * ho:2, j:j + 2 * ho:2].reshape(n, ho * ho)
             for i in range(7) for j in range(7) for c in range(x.shape[1])]
    at = jnp.stack(slabs, axis=1)
    co = w2d.shape[1]
    flops = 2 * n * ho * ho * kr * co
    bytes_accessed = kr * n * ho * ho * 2 + kr * co * 2 + n * ho * ho * co * 2
    return pl.pallas_call(
        functools.partial(_conv1_body, ho=ho, wo=ho),
        out_shape=jax.ShapeDtypeStruct((n, ho, ho, co), jnp.bfloat16),
        grid=(n,),
        in_specs=[
            pl.BlockSpec((1, kr, ho * ho), lambda i: (i, 0, 0)),
            pl.BlockSpec((kr, co), lambda i: (0, 0)),
            pl.BlockSpec((1, co), lambda i: (0, 0)),
            pl.BlockSpec((1, co), lambda i: (0, 0)),
        ],
        out_specs=pl.BlockSpec((1, ho, ho, co), lambda i: (i, 0, 0, 0)),
        compiler_params=pltpu.CompilerParams(
            dimension_semantics=("parallel",),
            vmem_limit_bytes=_VMEM_LIMIT),
        cost_estimate=pl.CostEstimate(flops=flops, transcendentals=0,
                                      bytes_accessed=bytes_accessed),
    )(at, w2d[:kr], scale, shift)


def _maxpool_body(e_ref, o_ref, out_ref, *, ho, wo):
    e = e_ref[0]
    o = o_ref[0]
    wm = jnp.maximum(jnp.maximum(e[:, 0:wo, :], o[:, 0:wo, :]),
                     e[:, 1:wo + 1, :])
    pa = wm[0:2 * ho].reshape(ho, 2, wo, wm.shape[-1])
    pb = wm[2:2 * ho + 2].reshape(ho, 2, wo, wm.shape[-1])
    out_ref[0] = jnp.maximum(jnp.maximum(pa[:, 0], pa[:, 1]), pb[:, 0])


def _maxpool_3x3_s2(x):
    n, h, w, c = x.shape
    ho, wo = h // 2, w // 2
    xp = jnp.pad(x, ((0, 0), (1, 1), (1, 1), (0, 0)))
    ev = xp[:, :, 0::2, :]
    od = xp[:, :, 1::2, :]
    hp, wp = h + 2, w // 2 + 1
    spec = pl.BlockSpec((1, hp, wp, c), lambda i: (i, 0, 0, 0))
    return pl.pallas_call(
        functools.partial(_maxpool_body, ho=ho, wo=wo),
        out_shape=jax.ShapeDtypeStruct((n, ho, wo, c), x.dtype),
        grid=(n,),
        in_specs=[spec, spec],
        out_specs=pl.BlockSpec((1, ho, wo, c), lambda i: (i, 0, 0, 0)),
        compiler_params=pltpu.CompilerParams(
            dimension_semantics=("parallel",),
            vmem_limit_bytes=_VMEM_LIMIT),
    )(ev, od)


def _tail_body(x_ref, w1_ref, b1_ref, w2_ref, b2_ref, w3_ref, b3_ref,
               w4_ref, b4_ref, out_ref, feat_ref, *, inv_hw):
    feats = jnp.sum(x_ref[...].astype(jnp.float32), axis=1) * inv_hw
    feat_ref[...] = feats
    h = feats
    for w_ref, b_ref in ((w1_ref, b1_ref), (w2_ref, b2_ref),
                         (w3_ref, b3_ref), (w4_ref, b4_ref)):
        h = jnp.dot(h.astype(w_ref.dtype), w_ref[...],
                    preferred_element_type=jnp.float32) + b_ref[...]
    ss = jnp.sum(h * h, axis=-1, keepdims=True)
    out_ref[...] = h * jax.lax.rsqrt(ss + 1e-12)


def _tail(x3, fcs, hw_real):
    n = x3.shape[0]
    c = x3.shape[-1]
    (w1, b1), (w2, b2), (w3, b3), (w4, b4) = fcs
    out, feats = pl.pallas_call(
        functools.partial(_tail_body, inv_hw=1.0 / float(hw_real)),
        out_shape=(jax.ShapeDtypeStruct((n, w4.shape[1]), jnp.float32),
                   jax.ShapeDtypeStruct((n, c), jnp.float32)),
        compiler_params=pltpu.CompilerParams(vmem_limit_bytes=_VMEM_LIMIT),
    )(x3, w1, b1, w2, b2, w3, b3, w4, b4)
    return out, feats


def _im2col3_s2(x, w_real):
    n, h, _, c = x.shape
    x = x[:, :, :w_real, :]
    ho, wo = h // 2, w_real // 2
    xp = jnp.pad(x, ((0, 0), (1, 1), (1, 1), (0, 0)))
    views = []
    for i in range(3):
        for j in range(3):
            views.append(xp[:, i:i + 2 * ho:2, j:j + 2 * wo:2, :])
    cols = jnp.stack(views, axis=3).reshape(n * ho * wo, 9 * c)
    return cols, ho, wo


def _to_padded_map(flat, n, ho, wo, co):
    x = flat.reshape(n, ho, wo, co)
    wp = _rnd_up(wo, 8)
    if wp != wo:
        x = jnp.pad(x, ((0, 0), (0, 0), (0, wp - wo), (0, 0)))
    return x


def kernel(x, conv1_w, bn1_scale, bn1_shift, l0b0_conv1_w, l0b0_bn1_scale, l0b0_bn1_shift, l0b0_conv2_w, l0b0_bn2_scale, l0b0_bn2_shift, l0b1_conv1_w, l0b1_bn1_scale, l0b1_bn1_shift, l0b1_conv2_w, l0b1_bn2_scale, l0b1_bn2_shift, l0b2_conv1_w, l0b2_bn1_scale, l0b2_bn1_shift, l0b2_conv2_w, l0b2_bn2_scale, l0b2_bn2_shift, l1b0_conv1_w, l1b0_bn1_scale, l1b0_bn1_shift, l1b0_conv2_w, l1b0_bn2_scale, l1b0_bn2_shift, l1b0_down_w, l1b0_down_scale, l1b0_down_shift, l1b1_conv1_w, l1b1_bn1_scale, l1b1_bn1_shift, l1b1_conv2_w, l1b1_bn2_scale, l1b1_bn2_shift, l1b2_conv1_w, l1b2_bn1_scale, l1b2_bn1_shift, l1b2_conv2_w, l1b2_bn2_scale, l1b2_bn2_shift, l1b3_conv1_w, l1b3_bn1_scale, l1b3_bn1_shift, l1b3_conv2_w, l1b3_bn2_scale, l1b3_bn2_shift, l2b0_conv1_w, l2b0_bn1_scale, l2b0_bn1_shift, l2b0_conv2_w, l2b0_bn2_scale, l2b0_bn2_shift, l2b0_down_w, l2b0_down_scale, l2b0_down_shift, l2b1_conv1_w, l2b1_bn1_scale, l2b1_bn1_shift, l2b1_conv2_w, l2b1_bn2_scale, l2b1_bn2_shift, l2b2_conv1_w, l2b2_bn1_scale, l2b2_bn1_shift, l2b2_conv2_w, l2b2_bn2_scale, l2b2_bn2_shift, l2b3_conv1_w, l2b3_bn1_scale, l2b3_bn1_shift, l2b3_conv2_w, l2b3_bn2_scale, l2b3_bn2_shift, l2b4_conv1_w, l2b4_bn1_scale, l2b4_bn1_shift, l2b4_conv2_w, l2b4_bn2_scale, l2b4_bn2_shift, l2b5_conv1_w, l2b5_bn1_scale, l2b5_bn1_shift, l2b5_conv2_w, l2b5_bn2_scale, l2b5_bn2_shift, l3b0_conv1_w, l3b0_bn1_scale, l3b0_bn1_shift, l3b0_conv2_w, l3b0_bn2_scale, l3b0_bn2_shift, l3b0_down_w, l3b0_down_scale, l3b0_down_shift, l3b1_conv1_w, l3b1_bn1_scale, l3b1_bn1_shift, l3b1_conv2_w, l3b1_bn2_scale, l3b1_bn2_shift, l3b2_conv1_w, l3b2_bn1_scale, l3b2_bn1_shift, l3b2_conv2_w, l3b2_bn2_scale, l3b2_bn2_shift, fc0_w, fc0_b, fc1_w, fc1_b, fc2_w, fc2_b, fc3_w, fc3_b):
    v = dict(locals())
    n = x.shape[0]

    h1 = _conv1_7x7_s2(x, conv1_w, bn1_scale, bn1_shift)
    cur = _maxpool_3x3_s2(h1)
    w_real = cur.shape[2]

    layer_cfg = ((3, 1), (4, 4), (6, 4), (3, 8))
    for li, (nb, bt) in enumerate(layer_cfg):
        for bi in range(nb):
            pfx = "l%db%d_" % (li, bi)
            w1, s1, c1 = v[pfx + "conv1_w"], v[pfx + "bn1_scale"], v[pfx + "bn1_shift"]
            w2, s2, c2 = v[pfx + "conv2_w"], v[pfx + "bn2_scale"], v[pfx + "bn2_shift"]
            if bi == 0 and li > 0:
                cin = cur.shape[-1]
                cols, ho, wo = _im2col3_s2(cur, w_real)
                co = w1.shape[1]
                b1_out = _matmul_bn(cols, w1, s1, c1, relu=True)
                b1_out = _to_padded_map(b1_out, n, ho, wo, co)
                xs = cur[:, ::2, :w_real:2, :].reshape(n * ho * wo, cin)
                idn = _matmul_bn(xs, v[pfx + "down_w"], v[pfx + "down_scale"],
                                 v[pfx + "down_shift"], relu=False)
                idn = _to_padded_map(idn, n, ho, wo, co)
                w_real = wo
            else:
                b1_out = _conv3x3_s1(cur, w1, s1, c1, w_real, relu=True, bt=bt)
                idn = cur
            cur = _conv3x3_s1(b1_out, w2, s2, c2, w_real, residual=idn,
                              relu=True, bt=bt)

    x3 = cur.reshape(n, cur.shape[1] * cur.shape[2], cur.shape[3])
    fcs = [(v["fc%d_w" % i], v["fc%d_b" % i]) for i in range(4)]
    out, feats = _tail(x3, fcs, hw_real=cur.shape[1] * w_real)
    return out[:, :4], feats


# speedup vs baseline: 3.1639x; 1.5333x over previous
import functools

import jax
import jax.numpy as jnp
from jax.experimental import pallas as pl
from jax.experimental.pallas import tpu as pltpu

_VMEM_LIMIT = 64 * 1024 * 1024


def _rnd_up(v, m):
    return ((v + m - 1) // m) * m


def _conv3s1_body(x_ref, w_ref, s_ref, c_ref, *rest, h, w_real, w_pad,
                  relu, has_res):
    if has_res:
        r_ref, o_ref, sh_ref, acc_ref = rest
    else:
        o_ref, sh_ref, acc_ref = rest
    bt = x_ref.shape[0]
    cin = x_ref.shape[-1]
    co = o_ref.shape[-1]
    m = bt * h * w_pad
    for dj in range(3):
        sh_ref[dj] = x_ref[:, :, dj:dj + w_pad, :].reshape(
            bt, (h + 2) * w_pad, cin)
    for di in range(3):
        for dj in range(3):
            a = sh_ref[dj, :, di * w_pad:(di + h) * w_pad, :].reshape(m, cin)
            wt = w_ref[(di * 3 + dj) * cin:(di * 3 + dj + 1) * cin, :]
            d = jnp.dot(a, wt, preferred_element_type=jnp.float32)
            if di == 0 and dj == 0:
                acc_ref[...] = d
            else:
                acc_ref[...] += d
    out = acc_ref[...] * s_ref[...] + c_ref[...]
    if has_res:
        out = out + r_ref[...].reshape(m, co).astype(jnp.float32)
    if relu:
        out = jnp.maximum(out, 0.0)
    if w_real != w_pad:
        col = jax.lax.broadcasted_iota(jnp.int32, (m, 1), 0) % w_pad
        out = jnp.where(col < w_real, out, 0.0)
    o_ref[...] = out.astype(o_ref.dtype).reshape(bt, h, w_pad, co)


def _conv3x3_s1(x, w2d, scale, shift, w_real, residual=None, relu=True, bt=1):
    n, h, w_pad, cin = x.shape
    kp, co = w2d.shape
    xp = jnp.pad(x, ((0, 0), (1, 1), (1, 1), (0, 0)))
    has_res = residual is not None
    in_specs = [
        pl.BlockSpec((bt, h + 2, w_pad + 2, cin), lambda i: (i, 0, 0, 0)),
        pl.BlockSpec((kp, co), lambda i: (0, 0)),
        pl.BlockSpec((1, co), lambda i: (0, 0)),
        pl.BlockSpec((1, co), lambda i: (0, 0)),
    ]
    args = [xp, w2d, scale, shift]
    if has_res:
        in_specs.append(pl.BlockSpec((bt, h, w_pad, co),
                                     lambda i: (i, 0, 0, 0)))
        args.append(residual)
    m = bt * h * w_pad
    flops = 2 * n * h * w_pad * kp * co
    bytes_accessed = (n * (h + 2) * (w_pad + 2) * cin * 2 + kp * co * 2
                      + n * h * w_pad * co * (4 if has_res else 2))
    return pl.pallas_call(
        functools.partial(_conv3s1_body, h=h, w_real=w_real, w_pad=w_pad,
                          relu=relu, has_res=has_res),
        out_shape=jax.ShapeDtypeStruct((n, h, w_pad, co), jnp.bfloat16),
        grid=(n // bt,),
        in_specs=in_specs,
        out_specs=pl.BlockSpec((bt, h, w_pad, co), lambda i: (i, 0, 0, 0)),
        scratch_shapes=[pltpu.VMEM((3, bt, (h + 2) * w_pad, cin),
                                   jnp.bfloat16),
                        pltpu.VMEM((m, co), jnp.float32)],
        compiler_params=pltpu.CompilerParams(
            dimension_semantics=("parallel",),
            vmem_limit_bytes=_VMEM_LIMIT),
        cost_estimate=pl.CostEstimate(flops=flops, transcendentals=0,
                                      bytes_accessed=bytes_accessed),
    )(*args)


def _mm_body(a_ref, b_ref, s_ref, c_ref, o_ref, *, relu):
    out = jnp.dot(a_ref[...], b_ref[...], preferred_element_type=jnp.float32)
    out = out * s_ref[...] + c_ref[...]
    if relu:
        out = jnp.maximum(out, 0.0)
    o_ref[...] = out.astype(o_ref.dtype)


def _matmul_bn(a, b, scale, shift, relu=True, tm=512):
    m, k = a.shape
    kb, nb = b.shape
    mp = _rnd_up(m, tm)
    if mp != m:
        a = jnp.pad(a, ((0, mp - m), (0, 0)))
    flops = 2 * mp * kb * nb
    bytes_accessed = mp * k * 2 + kb * nb * 2 + mp * nb * 2
    out = pl.pallas_call(
        functools.partial(_mm_body, relu=relu),
        out_shape=jax.ShapeDtypeStruct((mp, nb), jnp.bfloat16),
        grid=(mp // tm,),
        in_specs=[
            pl.BlockSpec((tm, k), lambda i: (i, 0)),
            pl.BlockSpec((kb, nb), lambda i: (0, 0)),
            pl.BlockSpec((1, nb), lambda i: (0, 0)),
            pl.BlockSpec((1, nb), lambda i: (0, 0)),
        ],
        out_specs=pl.BlockSpec((tm, nb), lambda i: (i, 0)),
        compiler_params=pltpu.CompilerParams(
            dimension_semantics=("parallel",),
            vmem_limit_bytes=_VMEM_LIMIT),
        cost_estimate=pl.CostEstimate(flops=flops, transcendentals=0,
                                      bytes_accessed=bytes_accessed),
    )(a, b, scale, shift)
    return out[:m] if mp != m else out


def _conv1_body(at_ref, b_ref, s_ref, c_ref, o_ref, *, ho, wo):
    kr = at_ref.shape[1]
    a = at_ref[0].reshape(kr, ho * wo)
    out = jax.lax.dot_general(a, b_ref[...], (((0,), (0,)), ((), ())),
                              preferred_element_type=jnp.float32)
    out = jnp.maximum(out * s_ref[...] + c_ref[...], 0.0)
    o_ref[...] = out.astype(o_ref.dtype).reshape(1, ho, wo, out.shape[-1])


def _conv1_7x7_s2(x, w2d, scale, shift):
    n = x.shape[0]
    hi = x.shape[2]
    ho = hi // 2
    kr = 7 * 7 * x.shape[1]
    xp = jnp.pad(x, ((0, 0), (0, 0), (3, 3), (3, 3))).astype(jnp.bfloat16)
    xs = [[xp[:, :, p::2, q::2] for q in range(2)] for p in range(2)]
    slabs = []
    for i in range(7):
        p, al = i % 2, i // 2
        for j in range(7):
            q, be = j % 2, j // 2
            for c in range(x.shape[1]):
                slabs.append(xs[p][q][:, c, al:al + ho, be:be + ho])
    at = jnp.stack(slabs, axis=1)
    co = w2d.shape[1]
    flops = 2 * n * ho * ho * kr * co
    bytes_accessed = kr * n * ho * ho * 2 + kr * co * 2 + n * ho * ho * co * 2
    return pl.pallas_call(
        functools.partial(_conv1_body, ho=ho, wo=ho),
        out_shape=jax.ShapeDtypeStruct((n, ho, ho, co), jnp.bfloat16),
        grid=(n,),
        in_specs=[
            pl.BlockSpec((1, kr, ho, ho), lambda i: (i, 0, 0, 0)),
            pl.BlockSpec((kr, co), lambda i: (0, 0)),
            pl.BlockSpec((1, co), lambda i: (0, 0)),
            pl.BlockSpec((1, co), lambda i: (0, 0)),
        ],
        out_specs=pl.BlockSpec((1, ho, ho, co), lambda i: (i, 0, 0, 0)),
        compiler_params=pltpu.CompilerParams(
            dimension_semantics=("parallel",),
            vmem_limit_bytes=_VMEM_LIMIT),
        cost_estimate=pl.CostEstimate(flops=flops, transcendentals=0,
                                      bytes_accessed=bytes_accessed),
    )(at, w2d[:kr], scale, shift)


def _maxpool_body(e_ref, o_ref, out_ref, *, ho, wo):
    e = e_ref[0]
    o = o_ref[0]
    wm = jnp.maximum(jnp.maximum(e[:, 0:wo, :], o[:, 0:wo, :]),
                     e[:, 1:wo + 1, :])
    pa = wm[0:2 * ho].reshape(ho, 2, wo, wm.shape[-1])
    pb = wm[2:2 * ho + 2].reshape(ho, 2, wo, wm.shape[-1])
    out_ref[0] = jnp.maximum(jnp.maximum(pa[:, 0], pa[:, 1]), pb[:, 0])


def _maxpool_3x3_s2(x):
    n, h, w, c = x.shape
    ho, wo = h // 2, w // 2
    xp = jnp.pad(x, ((0, 0), (1, 1), (1, 1), (0, 0)))
    ev = xp[:, :, 0::2, :]
    od = xp[:, :, 1::2, :]
    hp, wp = h + 2, w // 2 + 1
    spec = pl.BlockSpec((1, hp, wp, c), lambda i: (i, 0, 0, 0))
    return pl.pallas_call(
        functools.partial(_maxpool_body, ho=ho, wo=wo),
        out_shape=jax.ShapeDtypeStruct((n, ho, wo, c), x.dtype),
        grid=(n,),
        in_specs=[spec, spec],
        out_specs=pl.BlockSpec((1, ho, wo, c), lambda i: (i, 0, 0, 0)),
        compiler_params=pltpu.CompilerParams(
            dimension_semantics=("parallel",),
            vmem_limit_bytes=_VMEM_LIMIT),
    )(ev, od)


def _tail_body(x_ref, w1_ref, b1_ref, w2_ref, b2_ref, w3_ref, b3_ref,
               w4_ref, b4_ref, out_ref, feat_ref, *, inv_hw):
    feats = jnp.sum(x_ref[...].astype(jnp.float32), axis=1) * inv_hw
    feat_ref[...] = feats
    h = feats
    for w_ref, b_ref in ((w1_ref, b1_ref), (w2_ref, b2_ref),
                         (w3_ref, b3_ref), (w4_ref, b4_ref)):
        h = jnp.dot(h.astype(w_ref.dtype), w_ref[...],
                    preferred_element_type=jnp.float32) + b_ref[...]
    ss = jnp.sum(h * h, axis=-1, keepdims=True)
    out_ref[...] = h * jax.lax.rsqrt(ss + 1e-12)


def _tail(x3, fcs, hw_real):
    n = x3.shape[0]
    c = x3.shape[-1]
    (w1, b1), (w2, b2), (w3, b3), (w4, b4) = fcs
    out, feats = pl.pallas_call(
        functools.partial(_tail_body, inv_hw=1.0 / float(hw_real)),
        out_shape=(jax.ShapeDtypeStruct((n, w4.shape[1]), jnp.float32),
                   jax.ShapeDtypeStruct((n, c), jnp.float32)),
        compiler_params=pltpu.CompilerParams(vmem_limit_bytes=_VMEM_LIMIT),
    )(x3, w1, b1, w2, b2, w3, b3, w4, b4)
    return out, feats


def _im2col3_s2(x, w_real):
    n, h, _, c = x.shape
    x = x[:, :, :w_real, :]
    ho, wo = h // 2, w_real // 2
    xp = jnp.pad(x, ((0, 0), (1, 1), (1, 1), (0, 0)))
    views = []
    for i in range(3):
        for j in range(3):
            views.append(xp[:, i:i + 2 * ho:2, j:j + 2 * wo:2, :])
    cols = jnp.stack(views, axis=3).reshape(n * ho * wo, 9 * c)
    return cols, ho, wo


def _to_padded_map(flat, n, ho, wo, co):
    x = flat.reshape(n, ho, wo, co)
    wp = _rnd_up(wo, 8)
    if wp != wo:
        x = jnp.pad(x, ((0, 0), (0, 0), (0, wp - wo), (0, 0)))
    return x


def kernel(x, conv1_w, bn1_scale, bn1_shift, l0b0_conv1_w, l0b0_bn1_scale, l0b0_bn1_shift, l0b0_conv2_w, l0b0_bn2_scale, l0b0_bn2_shift, l0b1_conv1_w, l0b1_bn1_scale, l0b1_bn1_shift, l0b1_conv2_w, l0b1_bn2_scale, l0b1_bn2_shift, l0b2_conv1_w, l0b2_bn1_scale, l0b2_bn1_shift, l0b2_conv2_w, l0b2_bn2_scale, l0b2_bn2_shift, l1b0_conv1_w, l1b0_bn1_scale, l1b0_bn1_shift, l1b0_conv2_w, l1b0_bn2_scale, l1b0_bn2_shift, l1b0_down_w, l1b0_down_scale, l1b0_down_shift, l1b1_conv1_w, l1b1_bn1_scale, l1b1_bn1_shift, l1b1_conv2_w, l1b1_bn2_scale, l1b1_bn2_shift, l1b2_conv1_w, l1b2_bn1_scale, l1b2_bn1_shift, l1b2_conv2_w, l1b2_bn2_scale, l1b2_bn2_shift, l1b3_conv1_w, l1b3_bn1_scale, l1b3_bn1_shift, l1b3_conv2_w, l1b3_bn2_scale, l1b3_bn2_shift, l2b0_conv1_w, l2b0_bn1_scale, l2b0_bn1_shift, l2b0_conv2_w, l2b0_bn2_scale, l2b0_bn2_shift, l2b0_down_w, l2b0_down_scale, l2b0_down_shift, l2b1_conv1_w, l2b1_bn1_scale, l2b1_bn1_shift, l2b1_conv2_w, l2b1_bn2_scale, l2b1_bn2_shift, l2b2_conv1_w, l2b2_bn1_scale, l2b2_bn1_shift, l2b2_conv2_w, l2b2_bn2_scale, l2b2_bn2_shift, l2b3_conv1_w, l2b3_bn1_scale, l2b3_bn1_shift, l2b3_conv2_w, l2b3_bn2_scale, l2b3_bn2_shift, l2b4_conv1_w, l2b4_bn1_scale, l2b4_bn1_shift, l2b4_conv2_w, l2b4_bn2_scale, l2b4_bn2_shift, l2b5_conv1_w, l2b5_bn1_scale, l2b5_bn1_shift, l2b5_conv2_w, l2b5_bn2_scale, l2b5_bn2_shift, l3b0_conv1_w, l3b0_bn1_scale, l3b0_bn1_shift, l3b0_conv2_w, l3b0_bn2_scale, l3b0_bn2_shift, l3b0_down_w, l3b0_down_scale, l3b0_down_shift, l3b1_conv1_w, l3b1_bn1_scale, l3b1_bn1_shift, l3b1_conv2_w, l3b1_bn2_scale, l3b1_bn2_shift, l3b2_conv1_w, l3b2_bn1_scale, l3b2_bn1_shift, l3b2_conv2_w, l3b2_bn2_scale, l3b2_bn2_shift, fc0_w, fc0_b, fc1_w, fc1_b, fc2_w, fc2_b, fc3_w, fc3_b):
    v = dict(locals())
    n = x.shape[0]

    h1 = _conv1_7x7_s2(x, conv1_w, bn1_scale, bn1_shift)
    cur = _maxpool_3x3_s2(h1)
    w_real = cur.shape[2]

    layer_cfg = ((3, 1), (4, 4), (6, 4), (3, 8))
    for li, (nb, bt) in enumerate(layer_cfg):
        for bi in range(nb):
            pfx = "l%db%d_" % (li, bi)
            w1, s1, c1 = v[pfx + "conv1_w"], v[pfx + "bn1_scale"], v[pfx + "bn1_shift"]
            w2, s2, c2 = v[pfx + "conv2_w"], v[pfx + "bn2_scale"], v[pfx + "bn2_shift"]
            if bi == 0 and li > 0:
                cin = cur.shape[-1]
                cols, ho, wo = _im2col3_s2(cur, w_real)
                co = w1.shape[1]
                b1_out = _matmul_bn(cols, w1, s1, c1, relu=True)
                b1_out = _to_padded_map(b1_out, n, ho, wo, co)
                xs = cur[:, ::2, :w_real:2, :].reshape(n * ho * wo, cin)
                idn = _matmul_bn(xs, v[pfx + "down_w"], v[pfx + "down_scale"],
                                 v[pfx + "down_shift"], relu=False)
                idn = _to_padded_map(idn, n, ho, wo, co)
                w_real = wo
            else:
                b1_out = _conv3x3_s1(cur, w1, s1, c1, w_real, relu=True, bt=bt)
                idn = cur
            cur = _conv3x3_s1(b1_out, w2, s2, c2, w_real, residual=idn,
                              relu=True, bt=bt)

    x3 = cur.reshape(n, cur.shape[1] * cur.shape[2], cur.shape[3])
    fcs = [(v["fc%d_w" % i], v["fc%d_b" % i]) for i in range(4)]
    out, feats = _tail(x3, fcs, hw_real=cur.shape[1] * w_real)
    return out[:, :4], feats
